# baseline (speedup 1.0000x reference)
import functools

import jax
import jax.numpy as jnp
from jax import lax
from jax.experimental import pallas as pl
from jax.experimental.pallas import tpu as pltpu
from jax.experimental.pallas import tpu_sc as plsc

MAX_LEN = 8192
HIDDEN_DIM = 1024

_NSC = 2
_ROWS_PER_SC = MAX_LEN // _NSC
_CHUNK = 512
_NCHUNK = _ROWS_PER_SC // _CHUNK
_NBUF = 3


def _scs_copy(table_hbm, out_hbm, buf, *sems):
    gsems, ssems = sems[:_NBUF], sems[_NBUF:]
    base = lax.axis_index("c") * _ROWS_PER_SC

    def gather(c):
        return pltpu.async_copy(
            table_hbm.at[pl.ds(base + c * _CHUNK, _CHUNK)],
            buf.at[c % _NBUF], gsems[c % _NBUF])

    gh = [None] * _NCHUNK
    sh = [None] * _NCHUNK
    for c in range(_NBUF):
        gh[c] = gather(c)
    for c in range(_NCHUNK):
        gh[c].wait()
        sh[c] = pltpu.async_copy(
            buf.at[c % _NBUF],
            out_hbm.at[pl.ds(base + c * _CHUNK, _CHUNK)], ssems[c % _NBUF])
        if c + _NBUF < _NCHUNK:
            sh[c].wait()
            gh[c + _NBUF] = gather(c + _NBUF)
    for c in range(_NCHUNK - _NBUF, _NCHUNK):
        sh[c].wait()


def kernel(seq_len, pos_embedding):
    del seq_len
    kern = functools.partial(
        pl.kernel,
        mesh=plsc.ScalarSubcoreMesh(axis_name="c", num_cores=_NSC),
        out_type=jax.ShapeDtypeStruct((MAX_LEN, HIDDEN_DIM), jnp.float32),
        scratch_types=[
            pltpu.VMEM_SHARED((_NBUF, _CHUNK, HIDDEN_DIM), jnp.float32),
        ] + [pltpu.SemaphoreType.DMA] * (2 * _NBUF),
    )(_scs_copy)
    return kern(pos_embedding)


# speedup vs baseline: 1.0524x; 1.0524x over previous
import functools

import jax
import jax.numpy as jnp
from jax import lax
from jax.experimental import pallas as pl
from jax.experimental.pallas import tpu as pltpu
from jax.experimental.pallas import tpu_sc as plsc

MAX_LEN = 8192
HIDDEN_DIM = 1024

_INFO = plsc.get_sparse_core_info()
_NC = _INFO.num_cores
_NS = _INFO.num_subcores
_NW = _NC * _NS
_B_PER_W = MAX_LEN // _NW
_CHUNK = 16
_NCHUNK = _B_PER_W // _CHUNK
_NBUF = 7


def _pos_encoding_kernel(pos_hbm, table_hbm, out_hbm, idx_v, rows_v, *sems):
    gsems, ssems = sems[:_NBUF], sems[_NBUF:]
    wid = lax.axis_index("s") * _NC + lax.axis_index("c")
    base = wid * _B_PER_W

    pltpu.sync_copy(pos_hbm.at[wid], idx_v)

    def gather(c):
        return pltpu.async_copy(table_hbm.at[idx_v.at[c]],
                                rows_v.at[c % _NBUF], gsems[c % _NBUF])

    gh = [None] * _NCHUNK
    sh = [None] * _NCHUNK
    for c in range(_NBUF):
        gh[c] = gather(c)
    for c in range(_NCHUNK):
        gh[c].wait()
        sh[c] = pltpu.async_copy(
            rows_v.at[c % _NBUF],
            out_hbm.at[pl.ds(base + c * _CHUNK, _CHUNK)], ssems[c % _NBUF])
        if c + _NBUF < _NCHUNK:
            sh[c].wait()
            gh[c + _NBUF] = gather(c + _NBUF)
    for c in range(_NCHUNK - _NBUF, _NCHUNK):
        sh[c].wait()


def kernel(seq_len, pos_embedding):
    positions = jnp.minimum(
        jnp.arange(MAX_LEN, dtype=jnp.int32),
        jnp.asarray(seq_len, jnp.int32) - 1,
    ).reshape(_NW, _NCHUNK, _CHUNK)
    kern = functools.partial(
        pl.kernel,
        mesh=plsc.VectorSubcoreMesh(core_axis_name="c", subcore_axis_name="s"),
        out_type=jax.ShapeDtypeStruct((MAX_LEN, HIDDEN_DIM), jnp.float32),
        scratch_types=[
            pltpu.VMEM((_NCHUNK, _CHUNK), jnp.int32),
            pltpu.VMEM((_NBUF, _CHUNK, HIDDEN_DIM), jnp.float32),
        ] + [pltpu.SemaphoreType.DMA] * (2 * _NBUF),
    )(_pos_encoding_kernel)
    return kern(positions, pos_embedding)
